# baseline (speedup 1.0000x reference)
import jax
import jax.numpy as jnp
from jax import lax
from jax.experimental import pallas as pl
from jax.experimental.pallas import tpu as pltpu

_TOP_N = 1000
_MAX_BOXES = 18
_NMS_THRESH = 0.5
_N_ANCHORS = 21600
_N_PAD = 21632
_B = 8
_INT_MIN = -(2**31)
_BIG = 2**31 - 1


def _key_to_score(k):
    bits = jnp.where(k >= 0, k, k ^ jnp.int32(0x7FFFFFFF))
    return lax.bitcast_convert_type(bits, jnp.float32)


def _nms_body(score_ref, dx_ref, dy_ref, dw_ref, dh_ref,
              xa_ref, ya_ref, wa_ref, ha_ref,
              ocx_ref, ocy_ref, ow_ref, oh_ref, os_ref,
              pk_ref, rk_ref, x1_ref, y1_ref, x2_ref, y2_ref, area_ref):
    score = score_ref[...]
    col = lax.broadcasted_iota(jnp.int32, (_B, _N_PAD), 1)
    valid = col < _N_ANCHORS

    bits = lax.bitcast_convert_type(score, jnp.int32)
    skey = jnp.where(bits >= 0, bits, bits ^ jnp.int32(0x7FFFFFFF))
    skey = jnp.where(valid, skey, _INT_MIN)
    pk_ref[...] = skey

    xa = xa_ref[...]
    ya = ya_ref[...]
    wa = wa_ref[...]
    ha = ha_ref[...]
    cx = dx_ref[...] * wa + xa
    cy = dy_ref[...] * ha + ya
    w = wa * jnp.exp(dw_ref[...])
    h = ha * jnp.exp(dh_ref[...])
    x1 = cx - w / 2.0
    y1 = cy - h / 2.0
    x2 = cx + w / 2.0
    y2 = cy + h / 2.0
    x1_ref[...] = x1
    y1_ref[...] = y1
    x2_ref[...] = x2
    y2_ref[...] = y2
    area_ref[...] = (x2 - x1) * (y2 - y1)

    ones_col = jnp.ones((_N_PAD, 1), jnp.float32)

    def count(mask):
        return jnp.dot(mask.astype(jnp.float32), ones_col,
                       preferred_element_type=jnp.float32).astype(jnp.int32)

    def tstep(i, t):
        b = 31 - i
        cand = t + (jnp.int32(1) << b)
        cnt = count(pk_ref[...] >= cand)
        return jnp.where(cnt >= _TOP_N, cand, t)

    thr = lax.fori_loop(0, 32, tstep, jnp.full((_B, 1), _INT_MIN, jnp.int32))

    c_gt = count(skey > thr)
    c_eq = count(skey == thr)
    need = _TOP_N - c_gt

    def istep(i, m):
        b = 14 - i
        test = m + (jnp.int32(1) << b) - 1
        sk = pk_ref[...]
        cnt = count((sk == thr) & (col <= test))
        return jnp.where(cnt < need, m + (jnp.int32(1) << b), m)

    any_tie = jnp.max(c_eq - need)
    mloop = lax.fori_loop(0, jnp.where(any_tie > 0, 15, 0), istep,
                          jnp.zeros((_B, 1), jnp.int32))
    mcut = jnp.where(c_eq == need, jnp.int32(_BIG), mloop)
    not_cand = (skey < thr) | ((skey == thr) & (col > mcut))
    ckey = jnp.where(not_cand, _INT_MIN, skey)
    pk_ref[...] = ckey
    rk_ref[...] = ckey

    iota18 = lax.broadcasted_iota(jnp.int32, (_B, _MAX_BOXES), 1)
    zeros18 = jnp.zeros((_B, _MAX_BOXES), jnp.float32)

    def gather_at(onehot, arr):
        return jnp.sum(jnp.where(onehot, arr, 0.0), axis=1, keepdims=True)

    def pick_step(t, carry):
        kx1, ky1, kx2, ky2, ks, nk = carry
        sk = pk_ref[...]
        mx = jnp.max(sk, axis=1, keepdims=True)
        exists = mx > _INT_MIN
        pickm = (sk == mx) & exists
        j = jnp.min(jnp.where(pickm, col, _BIG), axis=1, keepdims=True)
        onehot = col == j
        x1v = x1_ref[...]
        y1v = y1_ref[...]
        x2v = x2_ref[...]
        y2v = y2_ref[...]
        gx1 = gather_at(onehot, x1v)
        gy1 = gather_at(onehot, y1v)
        gx2 = gather_at(onehot, x2v)
        gy2 = gather_at(onehot, y2v)
        xx1 = jnp.maximum(gx1, x1v)
        yy1 = jnp.maximum(gy1, y1v)
        xx2 = jnp.minimum(gx2, x2v)
        yy2 = jnp.minimum(gy2, y2v)
        ww = jnp.maximum(0.0, xx2 - xx1)
        hh = jnp.maximum(0.0, yy2 - yy1)
        ov = ww * hh / area_ref[...]
        dead = exists & ((ov > _NMS_THRESH) | onehot)
        pk_ref[...] = jnp.where(dead, _INT_MIN, sk)
        slotm = (iota18 == t) & exists
        kx1 = jnp.where(slotm, gx1, kx1)
        ky1 = jnp.where(slotm, gy1, ky1)
        kx2 = jnp.where(slotm, gx2, kx2)
        ky2 = jnp.where(slotm, gy2, ky2)
        ks = jnp.where(slotm, _key_to_score(mx), ks)
        nk = nk + exists.astype(jnp.int32)
        return kx1, ky1, kx2, ky2, ks, nk

    init = (zeros18, zeros18, zeros18, zeros18, zeros18,
            jnp.zeros((_B, 1), jnp.int32))
    kx1, ky1, kx2, ky2, ks, nk = lax.fori_loop(0, _MAX_BOXES, pick_step, init)

    def rank_step(rr, carry):
        px1, py1, px2, py2, psc = carry
        sk = rk_ref[...]
        mx = jnp.max(sk, axis=1, keepdims=True)
        exists = mx > _INT_MIN
        pickm = (sk == mx) & exists
        j = jnp.min(jnp.where(pickm, col, _BIG), axis=1, keepdims=True)
        onehot = col == j
        rk_ref[...] = jnp.where(onehot, _INT_MIN, sk)
        slotm = iota18 == (nk + rr)
        px1 = jnp.where(slotm, gather_at(onehot, x1_ref[...]), px1)
        py1 = jnp.where(slotm, gather_at(onehot, y1_ref[...]), py1)
        px2 = jnp.where(slotm, gather_at(onehot, x2_ref[...]), px2)
        py2 = jnp.where(slotm, gather_at(onehot, y2_ref[...]), py2)
        psc = jnp.where(slotm, _key_to_score(mx), psc)
        return px1, py1, px2, py2, psc

    keptm = iota18 < nk
    rinit = (jnp.where(keptm, kx1, 0.0), jnp.where(keptm, ky1, 0.0),
             jnp.where(keptm, kx2, 0.0), jnp.where(keptm, ky2, 0.0),
             jnp.where(keptm, ks, 0.0))
    n_pad_slots = _MAX_BOXES - jnp.min(nk)
    fx1, fy1, fx2, fy2, fsc = lax.fori_loop(0, n_pad_slots, rank_step, rinit)

    ocx_ref[...] = (fx1 + fx2) * 0.5
    ocy_ref[...] = (fy1 + fy2) * 0.5
    ow_ref[...] = fx2 - fx1
    oh_ref[...] = fy2 - fy1
    os_ref[...] = fsc


def _pallas_nms(score, dx, dy, dw, dh, xa, ya, wa, ha):
    out_sds = [jax.ShapeDtypeStruct((_B, _MAX_BOXES), jnp.float32)] * 5
    f32v = lambda: pltpu.VMEM((_B, _N_PAD), jnp.float32)
    i32v = lambda: pltpu.VMEM((_B, _N_PAD), jnp.int32)
    return pl.pallas_call(
        _nms_body,
        out_shape=out_sds,
        scratch_shapes=[i32v(), i32v(), f32v(), f32v(), f32v(), f32v(),
                        f32v()],
    )(score, dx, dy, dw, dh, xa, ya, wa, ha)


def kernel(x, anchor_boxes):
    npad = _N_PAD - _N_ANCHORS

    def pad_x(a):
        return jnp.pad(a, ((0, 0), (0, npad)))

    score = pad_x(x[:, :, 0])
    dx = pad_x(x[:, :, 2])
    dy = pad_x(x[:, :, 3])
    dw = pad_x(x[:, :, 4])
    dh = pad_x(x[:, :, 5])
    anc = jnp.transpose(anchor_boxes, (1, 0, 2, 3)).reshape(_N_ANCHORS, 4)

    def pad_a(a):
        return jnp.pad(a, (0, npad)).reshape(1, _N_PAD)

    xa = pad_a(anc[:, 0])
    ya = pad_a(anc[:, 1])
    wa = pad_a(anc[:, 2])
    ha = pad_a(anc[:, 3])
    ocx, ocy, ow, oh, osc = _pallas_nms(score, dx, dy, dw, dh, xa, ya, wa, ha)
    return jnp.stack([ocx, ocy, ow, oh, osc], axis=-1)


# speedup vs baseline: 1.3355x; 1.3355x over previous
import jax
import jax.numpy as jnp
from jax import lax
from jax.experimental import pallas as pl
from jax.experimental.pallas import tpu as pltpu

_TOP_N = 1000
_MAX_BOXES = 18
_NMS_THRESH = 0.5
_N_ANCHORS = 21600
_N_PAD = 21632
_B = 8
_INT_MIN = -(2**31)
_BIG = 2**31 - 1


def _key_to_score(k):
    bits = jnp.where(k >= 0, k, k ^ jnp.int32(0x7FFFFFFF))
    return lax.bitcast_convert_type(bits, jnp.float32)


def _nms_body(score_ref, dx_ref, dy_ref, dw_ref, dh_ref,
              xa_ref, ya_ref, wa_ref, ha_ref,
              ocx_ref, ocy_ref, ow_ref, oh_ref, os_ref,
              pk_ref, rk_ref, x1_ref, y1_ref, x2_ref, y2_ref, area_ref):
    score = score_ref[...]
    col = lax.broadcasted_iota(jnp.int32, (_B, _N_PAD), 1)
    valid = col < _N_ANCHORS

    bits = lax.bitcast_convert_type(score, jnp.int32)
    skey = jnp.where(bits >= 0, bits, bits ^ jnp.int32(0x7FFFFFFF))
    skey = jnp.where(valid, skey, _INT_MIN)
    pk_ref[...] = skey

    xa = xa_ref[...]
    ya = ya_ref[...]
    wa = wa_ref[...]
    ha = ha_ref[...]
    cx = dx_ref[...] * wa + xa
    cy = dy_ref[...] * ha + ya
    w = wa * jnp.exp(dw_ref[...])
    h = ha * jnp.exp(dh_ref[...])
    x1 = cx - w / 2.0
    y1 = cy - h / 2.0
    x2 = cx + w / 2.0
    y2 = cy + h / 2.0
    x1_ref[...] = x1
    y1_ref[...] = y1
    x2_ref[...] = x2
    y2_ref[...] = y2
    area = (x2 - x1) * (y2 - y1)
    u = area * jnp.float32(0.5 + 2.0**-25)
    ub = lax.bitcast_convert_type(u, jnp.int32)
    up = lax.bitcast_convert_type(ub + 1, jnp.float32)
    um = lax.bitcast_convert_type(ub - 1, jnp.float32)
    d_up = (up / area) <= _NMS_THRESH
    d_u = (u / area) <= _NMS_THRESH
    pthr = jnp.where(d_up, up, jnp.where(d_u, u, um))
    area_ref[...] = jnp.where((area == 0.0) | (u == 0.0), 0.0, pthr)

    def count(mask):
        return jnp.sum(mask.astype(jnp.int32), axis=1, keepdims=True)

    def tstep(i, t):
        b = 31 - i
        cand = t + (jnp.int32(1) << b)
        cnt = count(pk_ref[...] >= cand)
        return jnp.where(cnt >= _TOP_N, cand, t)

    thr = lax.fori_loop(0, 32, tstep, jnp.full((_B, 1), _INT_MIN, jnp.int32))

    c_gt = count(skey > thr)
    c_eq = count(skey == thr)
    need = _TOP_N - c_gt

    def istep(i, m):
        b = 14 - i
        test = m + (jnp.int32(1) << b) - 1
        sk = pk_ref[...]
        cnt = count((sk == thr) & (col <= test))
        return jnp.where(cnt < need, m + (jnp.int32(1) << b), m)

    any_tie = jnp.max(c_eq - need)
    mloop = lax.fori_loop(0, jnp.where(any_tie > 0, 15, 0), istep,
                          jnp.zeros((_B, 1), jnp.int32))
    mcut = jnp.where(c_eq == need, jnp.int32(_BIG), mloop)
    not_cand = (skey < thr) | ((skey == thr) & (col > mcut))
    ckey = jnp.where(not_cand, _INT_MIN, skey)
    pk_ref[...] = ckey
    rk_ref[...] = ckey

    iota18 = lax.broadcasted_iota(jnp.int32, (_B, _MAX_BOXES), 1)
    zeros18 = jnp.zeros((_B, _MAX_BOXES), jnp.float32)

    def gather_at(onehot, arr):
        return jnp.sum(jnp.where(onehot, arr, 0.0), axis=1, keepdims=True)

    def pick_step(t, carry):
        kx1, ky1, kx2, ky2, ks, nk = carry
        sk = pk_ref[...]
        mx = jnp.max(sk, axis=1, keepdims=True)
        exists = mx > _INT_MIN
        pickm = (sk == mx) & exists
        j = jnp.min(jnp.where(pickm, col, _BIG), axis=1, keepdims=True)
        onehot = col == j
        x1v = x1_ref[...]
        y1v = y1_ref[...]
        x2v = x2_ref[...]
        y2v = y2_ref[...]
        gx1 = gather_at(onehot, x1v)
        gy1 = gather_at(onehot, y1v)
        gx2 = gather_at(onehot, x2v)
        gy2 = gather_at(onehot, y2v)
        xx1 = jnp.maximum(gx1, x1v)
        yy1 = jnp.maximum(gy1, y1v)
        xx2 = jnp.minimum(gx2, x2v)
        yy2 = jnp.minimum(gy2, y2v)
        ww = jnp.maximum(0.0, xx2 - xx1)
        hh = jnp.maximum(0.0, yy2 - yy1)
        dead = exists & ((ww * hh > area_ref[...]) | onehot)
        pk_ref[...] = jnp.where(dead, _INT_MIN, sk)
        slotm = (iota18 == t) & exists
        kx1 = jnp.where(slotm, gx1, kx1)
        ky1 = jnp.where(slotm, gy1, ky1)
        kx2 = jnp.where(slotm, gx2, kx2)
        ky2 = jnp.where(slotm, gy2, ky2)
        ks = jnp.where(slotm, _key_to_score(mx), ks)
        nk = nk + exists.astype(jnp.int32)
        return kx1, ky1, kx2, ky2, ks, nk

    init = (zeros18, zeros18, zeros18, zeros18, zeros18,
            jnp.zeros((_B, 1), jnp.int32))
    kx1, ky1, kx2, ky2, ks, nk = lax.fori_loop(0, _MAX_BOXES, pick_step, init)

    def rank_step(rr, carry):
        px1, py1, px2, py2, psc = carry
        sk = rk_ref[...]
        mx = jnp.max(sk, axis=1, keepdims=True)
        exists = mx > _INT_MIN
        pickm = (sk == mx) & exists
        j = jnp.min(jnp.where(pickm, col, _BIG), axis=1, keepdims=True)
        onehot = col == j
        rk_ref[...] = jnp.where(onehot, _INT_MIN, sk)
        slotm = iota18 == (nk + rr)
        px1 = jnp.where(slotm, gather_at(onehot, x1_ref[...]), px1)
        py1 = jnp.where(slotm, gather_at(onehot, y1_ref[...]), py1)
        px2 = jnp.where(slotm, gather_at(onehot, x2_ref[...]), px2)
        py2 = jnp.where(slotm, gather_at(onehot, y2_ref[...]), py2)
        psc = jnp.where(slotm, _key_to_score(mx), psc)
        return px1, py1, px2, py2, psc

    keptm = iota18 < nk
    rinit = (jnp.where(keptm, kx1, 0.0), jnp.where(keptm, ky1, 0.0),
             jnp.where(keptm, kx2, 0.0), jnp.where(keptm, ky2, 0.0),
             jnp.where(keptm, ks, 0.0))
    n_pad_slots = _MAX_BOXES - jnp.min(nk)
    fx1, fy1, fx2, fy2, fsc = lax.fori_loop(0, n_pad_slots, rank_step, rinit)

    ocx_ref[...] = (fx1 + fx2) * 0.5
    ocy_ref[...] = (fy1 + fy2) * 0.5
    ow_ref[...] = fx2 - fx1
    oh_ref[...] = fy2 - fy1
    os_ref[...] = fsc


def _pallas_nms(score, dx, dy, dw, dh, xa, ya, wa, ha):
    out_sds = [jax.ShapeDtypeStruct((_B, _MAX_BOXES), jnp.float32)] * 5
    f32v = lambda: pltpu.VMEM((_B, _N_PAD), jnp.float32)
    i32v = lambda: pltpu.VMEM((_B, _N_PAD), jnp.int32)
    return pl.pallas_call(
        _nms_body,
        out_shape=out_sds,
        scratch_shapes=[i32v(), i32v(), f32v(), f32v(), f32v(), f32v(),
                        f32v()],
    )(score, dx, dy, dw, dh, xa, ya, wa, ha)


def kernel(x, anchor_boxes):
    npad = _N_PAD - _N_ANCHORS

    def pad_x(a):
        return jnp.pad(a, ((0, 0), (0, npad)))

    score = pad_x(x[:, :, 0])
    dx = pad_x(x[:, :, 2])
    dy = pad_x(x[:, :, 3])
    dw = pad_x(x[:, :, 4])
    dh = pad_x(x[:, :, 5])
    anc = jnp.transpose(anchor_boxes, (1, 0, 2, 3)).reshape(_N_ANCHORS, 4)

    def pad_a(a):
        return jnp.pad(a, (0, npad)).reshape(1, _N_PAD)

    xa = pad_a(anc[:, 0])
    ya = pad_a(anc[:, 1])
    wa = pad_a(anc[:, 2])
    ha = pad_a(anc[:, 3])
    ocx, ocy, ow, oh, osc = _pallas_nms(score, dx, dy, dw, dh, xa, ya, wa, ha)
    return jnp.stack([ocx, ocy, ow, oh, osc], axis=-1)
